# Initial kernel scaffold; baseline (speedup 1.0000x reference)
import jax
import jax.numpy as jnp
from jax.experimental import pallas as pl


def kernel(Z, leq0, leq1, leq2):
    raise NotImplementedError("write your pallas kernel here")


# speedup vs baseline: 4.0706x; 4.0706x over previous
import functools

import jax
import jax.numpy as jnp
from jax import lax
from jax.experimental import pallas as pl
from jax.experimental.pallas import tpu as pltpu
from jax.experimental.pallas import tpu_sc as plsc

_F = 64
_NSPECIES = 14
_DIMS = (1, 3, 5)


@functools.partial(jax.jit, static_argnums=())
def _gather3(Z, t0, t1, t2):
    N = Z.shape[0]
    info = plsc.get_sparse_core_info()
    nc, ns = info.num_cores, info.num_subcores
    nw = nc * ns
    bw = N // nw
    C = 64
    nch = bw // C
    d0, d1, d2 = (_F * k for k in _DIMS)

    @functools.partial(
        pl.kernel,
        mesh=plsc.VectorSubcoreMesh(core_axis_name="c", subcore_axis_name="s"),
        compiler_params=pltpu.CompilerParams(use_tc_tiling_on_sc=False),
        out_type=[
            jax.ShapeDtypeStruct((N, d0), jnp.float32),
            jax.ShapeDtypeStruct((N, d1), jnp.float32),
            jax.ShapeDtypeStruct((N, d2), jnp.float32),
        ],
        scratch_types=[
            pltpu.VMEM((nch, C), jnp.int32),
            pltpu.VMEM((C, d0), jnp.float32),
            pltpu.VMEM((C, d1), jnp.float32),
            pltpu.VMEM((C, d2), jnp.float32),
            pltpu.SemaphoreType.DMA,
        ],
    )
    def k(z_hbm, t0_hbm, t1_hbm, t2_hbm, o0_hbm, o1_hbm, o2_hbm,
          idx_v, r0, r1, r2, sem):
        wid = lax.axis_index("s") * nc + lax.axis_index("c")
        pltpu.sync_copy(z_hbm.at[wid], idx_v)

        def body(ci, carry):
            idx = idx_v.at[ci]
            cp0 = pltpu.async_copy(t0_hbm.at[idx], r0, sem)
            cp1 = pltpu.async_copy(t1_hbm.at[idx], r1, sem)
            cp2 = pltpu.async_copy(t2_hbm.at[idx], r2, sem)
            cp0.wait()
            cp1.wait()
            cp2.wait()
            off = wid * bw + ci * C
            pltpu.sync_copy(r0, o0_hbm.at[pl.ds(off, C)])
            pltpu.sync_copy(r1, o1_hbm.at[pl.ds(off, C)])
            pltpu.sync_copy(r2, o2_hbm.at[pl.ds(off, C)])
            return carry

        lax.fori_loop(0, nch, body, 0)

    return k(Z.reshape(nw, nch, C), t0, t1, t2)


def kernel(Z, leq0, leq1, leq2):
    N = Z.shape[0]
    t0 = leq0.reshape(_NSPECIES, _F * _DIMS[0])
    t1 = leq1.reshape(_NSPECIES, _F * _DIMS[1])
    t2 = leq2.reshape(_NSPECIES, _F * _DIMS[2])
    o0, o1, o2 = _gather3(Z.astype(jnp.int32), t0, t1, t2)
    return (
        o0.reshape(N, _F, 1),
        o1.reshape(N, _F, 3),
        o2.reshape(N, _F, 5),
        jnp.zeros((N, _F, 7), jnp.float32),
        jnp.zeros((N, _F, 9), jnp.float32),
        jnp.zeros((N, _F, 11), jnp.float32),
    )
